# initial kernel scaffold (unmeasured)
import jax
import jax.numpy as jnp
from jax import lax
from jax.experimental import pallas as pl
from jax.experimental.pallas import tpu as pltpu

N_DEV = 4
N_TOK = 2048
D = 1024
E_LOC = 8
E_TOT = 32
CAP = 204.0


def kernel(x, router_W, route_idx, expert_W):
    del router_W

    oh = (route_idx[:, 0:1] == jnp.arange(E_TOT, dtype=jnp.int32)[None, :])
    oh = oh.astype(jnp.float32)
    cum = jnp.cumsum(oh, axis=0) - oh
    meta = jnp.stack(
        [
            jnp.concatenate(
                [oh[:, d * E_LOC:(d + 1) * E_LOC],
                 cum[:, d * E_LOC:(d + 1) * E_LOC]], axis=1)
            for d in range(N_DEV)
        ],
        axis=0,
    )
    xb = x.astype(jnp.bfloat16)
    wb = expert_W.astype(jnp.bfloat16)

    def body(xb_ref, meta_ref, w_hbm, out_ref,
             xg, mrecv, precv, acc, wvm,
             m_snd, m_rcv, h_snd, h_rcv, p_snd, p_rcv, w_sem):
        me = lax.axis_index("i")
        left = lax.rem(me + N_DEV - 1, N_DEV)
        right = lax.rem(me + 1, N_DEV)

        bsem = pltpu.get_barrier_semaphore()
        for nbr in (left, right):
            pl.semaphore_signal(bsem, inc=1, device_id=(nbr,),
                                device_id_type=pl.DeviceIdType.MESH)
        pl.semaphore_wait(bsem, 2)

        meta_rdmas = []
        for j in (1, 2, 3):
            d = lax.rem(me + j, N_DEV)
            r = pltpu.make_async_remote_copy(
                src_ref=meta_ref.at[d],
                dst_ref=mrecv.at[3 - j],
                send_sem=m_snd.at[j - 1],
                recv_sem=m_rcv.at[3 - j],
                device_id=(d,),
                device_id_type=pl.DeviceIdType.MESH,
            )
            r.start()
            meta_rdmas.append(r)

        hop_rdmas = []
        h0 = pltpu.make_async_remote_copy(
            src_ref=xb_ref,
            dst_ref=xg.at[0],
            send_sem=h_snd.at[0],
            recv_sem=h_rcv.at[0],
            device_id=(right,),
            device_id_type=pl.DeviceIdType.MESH,
        )
        h0.start()
        hop_rdmas.append(h0)

        for r in meta_rdmas:
            r.wait_recv()

        c_own = jnp.sum(meta_ref[me][:, 0:E_LOC], axis=0)
        shard_ids = [me]
        shard_cnt = [c_own]
        for q in range(3):
            shard_ids.append(lax.rem(me + q + 1, N_DEV))
            shard_cnt.append(jnp.sum(mrecv[q][:, 0:E_LOC], axis=0))

        def mask_for(origin, blk):
            base = jnp.zeros((E_LOC,), jnp.float32)
            for sid, cnt in zip(shard_ids, shard_cnt):
                base = base + jnp.where(sid < origin, 1.0, 0.0) * cnt
            keep = (blk[:, E_LOC:2 * E_LOC] + base[None, :]) < CAP
            return jnp.where(keep, blk[:, 0:E_LOC], 0.0).astype(jnp.bfloat16)

        def compute_partial(xo, mask, slot):
            cp = pltpu.make_async_copy(w_hbm.at[0], wvm.at[0], w_sem.at[0])
            cp.start()
            for le in range(E_LOC):
                cur = le % 2
                pltpu.make_async_copy(
                    w_hbm.at[le], wvm.at[cur], w_sem.at[cur]).wait()
                if le < E_LOC - 1:
                    pltpu.make_async_copy(
                        w_hbm.at[le + 1], wvm.at[(le + 1) % 2],
                        w_sem.at[(le + 1) % 2]).start()
                y = jnp.dot(xo, wvm[cur],
                            preferred_element_type=jnp.bfloat16)
                contrib = y * mask[:, le:le + 1]
                if le == 0:
                    acc[slot] = contrib
                else:
                    acc[slot] = acc[slot] + contrib

        own_mask = mask_for(me, meta_ref[me])
        compute_partial(xb_ref[...], own_mask, 0)
        out_ref[...] = acc[0].astype(jnp.float32)

        part_rdmas = {}
        for r in (1, 2, 3):
            hop = r - 1
            pltpu.make_async_remote_copy(
                src_ref=xg.at[hop], dst_ref=xg.at[hop],
                send_sem=h_snd.at[hop], recv_sem=h_rcv.at[hop],
                device_id=(left,), device_id_type=pl.DeviceIdType.MESH,
            ).wait_recv()
            if r <= 2:
                hf = pltpu.make_async_remote_copy(
                    src_ref=xg.at[hop],
                    dst_ref=xg.at[r],
                    send_sem=h_snd.at[r],
                    recv_sem=h_rcv.at[r],
                    device_id=(right,),
                    device_id_type=pl.DeviceIdType.MESH,
                )
                hf.start()
                hop_rdmas.append(hf)

            origin = lax.rem(me - r + N_DEV, N_DEV)
            slot = r % 2
            if r == 3:
                part_rdmas[1].wait_send()
            compute_partial(xg[hop], mask_for(origin, mrecv[3 - r]), slot)
            pr = pltpu.make_async_remote_copy(
                src_ref=acc.at[slot],
                dst_ref=precv.at[r - 1],
                send_sem=p_snd.at[r - 1],
                recv_sem=p_rcv.at[r - 1],
                device_id=(origin,),
                device_id_type=pl.DeviceIdType.MESH,
            )
            pr.start()
            part_rdmas[r] = pr

        for q in range(3):
            pltpu.make_async_remote_copy(
                src_ref=acc.at[0], dst_ref=precv.at[q],
                send_sem=p_snd.at[q], recv_sem=p_rcv.at[q],
                device_id=(left,), device_id_type=pl.DeviceIdType.MESH,
            ).wait_recv()
            out_ref[...] = out_ref[...] + precv[q].astype(jnp.float32)

        for r in meta_rdmas:
            r.wait_send()
        for r in hop_rdmas:
            r.wait_send()
        part_rdmas[2].wait_send()
        part_rdmas[3].wait_send()

    return pl.pallas_call(
        body,
        out_shape=jax.ShapeDtypeStruct((N_TOK, D), jnp.float32),
        in_specs=[
            pl.BlockSpec(memory_space=pltpu.VMEM),
            pl.BlockSpec(memory_space=pltpu.VMEM),
            pl.BlockSpec(memory_space=pltpu.ANY),
        ],
        out_specs=pl.BlockSpec(memory_space=pltpu.VMEM),
        scratch_shapes=[
            pltpu.VMEM((3, N_TOK, D), jnp.bfloat16),
            pltpu.VMEM((3, N_TOK, 2 * E_LOC), jnp.float32),
            pltpu.VMEM((3, N_TOK, D), jnp.bfloat16),
            pltpu.VMEM((2, N_TOK, D), jnp.bfloat16),
            pltpu.VMEM((2, D, D), jnp.bfloat16),
            pltpu.SemaphoreType.DMA((3,)),
            pltpu.SemaphoreType.DMA((3,)),
            pltpu.SemaphoreType.DMA((3,)),
            pltpu.SemaphoreType.DMA((3,)),
            pltpu.SemaphoreType.DMA((3,)),
            pltpu.SemaphoreType.DMA((3,)),
            pltpu.SemaphoreType.DMA((2,)),
        ],
        compiler_params=pltpu.CompilerParams(collective_id=0),
    )(xb, meta, wb)


# baseline (device time: 304683 ns/iter reference)
import jax
import jax.numpy as jnp
from jax import lax
from jax.experimental import pallas as pl
from jax.experimental.pallas import tpu as pltpu

N_DEV = 4
N_TOK = 2048
D = 1024
E_LOC = 8
E_TOT = 32
CAP = 204.0


def kernel(x, router_W, route_idx, expert_W):
    del router_W

    oh = (route_idx[:, 0:1] == jnp.arange(E_TOT, dtype=jnp.int32)[None, :])
    oh = oh.astype(jnp.float32)
    cum = jnp.cumsum(oh, axis=0) - oh
    meta = jnp.stack(
        [
            jnp.concatenate(
                [oh[:, d * E_LOC:(d + 1) * E_LOC],
                 cum[:, d * E_LOC:(d + 1) * E_LOC]], axis=1)
            for d in range(N_DEV)
        ],
        axis=0,
    )
    xb = x.astype(jnp.bfloat16)
    wb = expert_W.astype(jnp.bfloat16)

    def body(xb_ref, meta_ref, w_hbm, out_ref,
             xg, mrecv, precv, acc, wvm,
             m_snd, m_rcv, h_snd, h_rcv, p_snd, p_rcv, w_sem):
        me = lax.axis_index("i")
        left = lax.rem(me + N_DEV - 1, N_DEV)
        right = lax.rem(me + 1, N_DEV)

        bsem = pltpu.get_barrier_semaphore()
        for nbr in (left, right):
            pl.semaphore_signal(bsem, inc=1, device_id=(nbr,),
                                device_id_type=pl.DeviceIdType.MESH)
        pl.semaphore_wait(bsem, 2)

        meta_rdmas = []
        for j in (1, 2, 3):
            d = lax.rem(me + j, N_DEV)
            r = pltpu.make_async_remote_copy(
                src_ref=meta_ref.at[d],
                dst_ref=mrecv.at[3 - j],
                send_sem=m_snd.at[j - 1],
                recv_sem=m_rcv.at[3 - j],
                device_id=(d,),
                device_id_type=pl.DeviceIdType.MESH,
            )
            r.start()
            meta_rdmas.append(r)

        hop_rdmas = []
        h0 = pltpu.make_async_remote_copy(
            src_ref=xb_ref,
            dst_ref=xg.at[0],
            send_sem=h_snd.at[0],
            recv_sem=h_rcv.at[0],
            device_id=(right,),
            device_id_type=pl.DeviceIdType.MESH,
        )
        h0.start()
        hop_rdmas.append(h0)

        for r in meta_rdmas:
            r.wait_recv()

        c_own = jnp.sum(meta_ref[me][:, 0:E_LOC], axis=0)
        shard_ids = [me]
        shard_cnt = [c_own]
        for q in range(3):
            shard_ids.append(lax.rem(me + q + 1, N_DEV))
            shard_cnt.append(jnp.sum(mrecv[q][:, 0:E_LOC], axis=0))

        def mask_for(origin, blk):
            base = jnp.zeros((E_LOC,), jnp.float32)
            for sid, cnt in zip(shard_ids, shard_cnt):
                base = base + jnp.where(sid < origin, 1.0, 0.0) * cnt
            keep = (blk[:, E_LOC:2 * E_LOC] + base[None, :]) < CAP
            return jnp.where(keep, blk[:, 0:E_LOC], 0.0).astype(jnp.bfloat16)

        def compute_partial(xo, mask, slot):
            cp = pltpu.make_async_copy(w_hbm.at[0], wvm.at[0], w_sem.at[0])
            cp.start()
            for le in range(E_LOC):
                cur = le % 2
                pltpu.make_async_copy(
                    w_hbm.at[le], wvm.at[cur], w_sem.at[cur]).wait()
                if le < E_LOC - 1:
                    pltpu.make_async_copy(
                        w_hbm.at[le + 1], wvm.at[(le + 1) % 2],
                        w_sem.at[(le + 1) % 2]).start()
                y = jnp.dot(xo, wvm[cur],
                            preferred_element_type=jnp.float32)
                contrib = y.astype(jnp.bfloat16) * mask[:, le:le + 1]
                if le == 0:
                    acc[slot] = contrib
                else:
                    acc[slot] = acc[slot] + contrib

        own_mask = mask_for(me, meta_ref[me])
        compute_partial(xb_ref[...], own_mask, 0)
        out_ref[...] = acc[0].astype(jnp.float32)

        part_rdmas = {}
        for r in (1, 2, 3):
            hop = r - 1
            pltpu.make_async_remote_copy(
                src_ref=xg.at[hop], dst_ref=xg.at[hop],
                send_sem=h_snd.at[hop], recv_sem=h_rcv.at[hop],
                device_id=(left,), device_id_type=pl.DeviceIdType.MESH,
            ).wait_recv()
            if r <= 2:
                hf = pltpu.make_async_remote_copy(
                    src_ref=xg.at[hop],
                    dst_ref=xg.at[r],
                    send_sem=h_snd.at[r],
                    recv_sem=h_rcv.at[r],
                    device_id=(right,),
                    device_id_type=pl.DeviceIdType.MESH,
                )
                hf.start()
                hop_rdmas.append(hf)

            origin = lax.rem(me - r + N_DEV, N_DEV)
            slot = r % 2
            if r == 3:
                part_rdmas[1].wait_send()
            compute_partial(xg[hop], mask_for(origin, mrecv[3 - r]), slot)
            pr = pltpu.make_async_remote_copy(
                src_ref=acc.at[slot],
                dst_ref=precv.at[r - 1],
                send_sem=p_snd.at[r - 1],
                recv_sem=p_rcv.at[r - 1],
                device_id=(origin,),
                device_id_type=pl.DeviceIdType.MESH,
            )
            pr.start()
            part_rdmas[r] = pr

        for q in range(3):
            pltpu.make_async_remote_copy(
                src_ref=acc.at[0], dst_ref=precv.at[q],
                send_sem=p_snd.at[q], recv_sem=p_rcv.at[q],
                device_id=(left,), device_id_type=pl.DeviceIdType.MESH,
            ).wait_recv()
            out_ref[...] = out_ref[...] + precv[q].astype(jnp.float32)

        for r in meta_rdmas:
            r.wait_send()
        for r in hop_rdmas:
            r.wait_send()
        part_rdmas[2].wait_send()
        part_rdmas[3].wait_send()

    return pl.pallas_call(
        body,
        out_shape=jax.ShapeDtypeStruct((N_TOK, D), jnp.float32),
        in_specs=[
            pl.BlockSpec(memory_space=pltpu.VMEM),
            pl.BlockSpec(memory_space=pltpu.VMEM),
            pl.BlockSpec(memory_space=pl.ANY),
        ],
        out_specs=pl.BlockSpec(memory_space=pltpu.VMEM),
        scratch_shapes=[
            pltpu.VMEM((3, N_TOK, D), jnp.bfloat16),
            pltpu.VMEM((3, N_TOK, 2 * E_LOC), jnp.float32),
            pltpu.VMEM((3, N_TOK, D), jnp.bfloat16),
            pltpu.VMEM((2, N_TOK, D), jnp.bfloat16),
            pltpu.VMEM((2, D, D), jnp.bfloat16),
            pltpu.SemaphoreType.DMA((3,)),
            pltpu.SemaphoreType.DMA((3,)),
            pltpu.SemaphoreType.DMA((3,)),
            pltpu.SemaphoreType.DMA((3,)),
            pltpu.SemaphoreType.DMA((3,)),
            pltpu.SemaphoreType.DMA((3,)),
            pltpu.SemaphoreType.DMA((2,)),
        ],
        compiler_params=pltpu.CompilerParams(
            collective_id=0, vmem_limit_bytes=64 * 1024 * 1024),
    )(xb, meta, wb)


# device time: 286167 ns/iter; 1.0647x vs baseline; 1.0647x over previous
import jax
import jax.numpy as jnp
from jax import lax
from jax.experimental import pallas as pl
from jax.experimental.pallas import tpu as pltpu

N_DEV = 4
N_TOK = 2048
D = 1024
E_LOC = 8
E_TOT = 32
CAP = 204.0


def kernel(x, router_W, route_idx, expert_W):
    del router_W

    oh = (route_idx[:, 0:1] == jnp.arange(E_TOT, dtype=jnp.int32)[None, :])
    oh = oh.astype(jnp.float32)
    cum = jnp.cumsum(oh, axis=0) - oh
    meta = jnp.stack(
        [
            jnp.concatenate(
                [oh[:, d * E_LOC:(d + 1) * E_LOC].T,
                 cum[:, d * E_LOC:(d + 1) * E_LOC].T], axis=0)
            for d in range(N_DEV)
        ],
        axis=0,
    )
    xb = x.astype(jnp.bfloat16)
    wb = expert_W.astype(jnp.bfloat16)

    def body(xb_ref, meta_ref, w_hbm, out_ref,
             xg, mrecv, precv, acc, wvm, pk,
             m_snd, m_rcv, h_snd, h_rcv, p_snd, p_rcv, w_sem):
        me = lax.axis_index("i")
        left = lax.rem(me + N_DEV - 1, N_DEV)
        right = lax.rem(me + 1, N_DEV)

        bsem = pltpu.get_barrier_semaphore()
        for nbr in (left, right):
            pl.semaphore_signal(bsem, inc=1, device_id=(nbr,),
                                device_id_type=pl.DeviceIdType.MESH)
        pl.semaphore_wait(bsem, 2)

        meta_rdmas = []
        for j in (1, 2, 3):
            d = lax.rem(me + j, N_DEV)
            r = pltpu.make_async_remote_copy(
                src_ref=meta_ref.at[d],
                dst_ref=mrecv.at[3 - j],
                send_sem=m_snd.at[j - 1],
                recv_sem=m_rcv.at[3 - j],
                device_id=(d,),
                device_id_type=pl.DeviceIdType.MESH,
            )
            r.start()
            meta_rdmas.append(r)

        hop_rdmas = []
        h0 = pltpu.make_async_remote_copy(
            src_ref=xb_ref,
            dst_ref=xg.at[0],
            send_sem=h_snd.at[0],
            recv_sem=h_rcv.at[0],
            device_id=(right,),
            device_id_type=pl.DeviceIdType.MESH,
        )
        h0.start()
        hop_rdmas.append(h0)

        for r in meta_rdmas:
            r.wait_recv()

        c_own = jnp.sum(meta_ref[me][0:E_LOC, :], axis=1)
        shard_ids = [me]
        shard_cnt = [c_own]
        for q in range(3):
            shard_ids.append(lax.rem(me + q + 1, N_DEV))
            shard_cnt.append(jnp.sum(mrecv[q][0:E_LOC, :], axis=1))

        def base_for(origin):
            base = jnp.zeros((E_LOC,), jnp.float32)
            for sid, cnt in zip(shard_ids, shard_cnt):
                base = base + jnp.where(sid < origin, 1.0, 0.0) * cnt
            return base

        CAP_PAD = 256
        slot_iota = lax.broadcasted_iota(jnp.int32, (CAP_PAD, N_TOK), 0)

        def compute_partial(blk, base, slot, xo_ref):
            pk[0] = blk[E_LOC:2 * E_LOC, :] + base[:, None]
            pk[1] = blk[0:E_LOC, :] * (pk[0] < CAP)
            acc[slot] = jnp.zeros((N_TOK, D), jnp.bfloat16)
            pltpu.make_async_copy(w_hbm.at[0], wvm.at[0], w_sem.at[0]).start()

            def le_body(le, _):
                cur = lax.rem(le, 2)
                pltpu.make_async_copy(
                    w_hbm.at[le], wvm.at[cur], w_sem.at[cur]).wait()

                @pl.when(le < E_LOC - 1)
                def _():
                    nxt = lax.rem(le + 1, 2)
                    pltpu.make_async_copy(
                        w_hbm.at[le + 1], wvm.at[nxt], w_sem.at[nxt]).start()

                pos_i = pk[0, le, :].astype(jnp.int32)
                keep = pk[1, le, :]
                g = jnp.where(
                    (slot_iota == pos_i[None, :]) & (keep[None, :] > 0),
                    1.0, 0.0).astype(jnp.bfloat16)
                xc = jnp.dot(g, xo_ref[...], preferred_element_type=jnp.float32)
                y = jnp.dot(xc.astype(jnp.bfloat16), wvm[cur],
                            preferred_element_type=jnp.float32)
                sc = lax.dot_general(
                    g, y.astype(jnp.bfloat16), (((0,), (0,)), ((), ())),
                    preferred_element_type=jnp.float32)
                acc[slot] = acc[slot] + sc.astype(jnp.bfloat16)
                return 0

            lax.fori_loop(0, E_LOC, le_body, 0)

        compute_partial(meta_ref[me], base_for(me), 0, xb_ref)
        out_ref[...] = acc[0].astype(jnp.float32)

        part_rdmas = {}
        for r in (1, 2, 3):
            hop = r - 1
            pltpu.make_async_remote_copy(
                src_ref=xg.at[hop], dst_ref=xg.at[hop],
                send_sem=h_snd.at[hop], recv_sem=h_rcv.at[hop],
                device_id=(left,), device_id_type=pl.DeviceIdType.MESH,
            ).wait_recv()
            if r <= 2:
                hf = pltpu.make_async_remote_copy(
                    src_ref=xg.at[hop],
                    dst_ref=xg.at[r],
                    send_sem=h_snd.at[r],
                    recv_sem=h_rcv.at[r],
                    device_id=(right,),
                    device_id_type=pl.DeviceIdType.MESH,
                )
                hf.start()
                hop_rdmas.append(hf)

            origin = lax.rem(me - r + N_DEV, N_DEV)
            slot = r % 2
            if r == 3:
                part_rdmas[1].wait_send()
            compute_partial(mrecv[3 - r], base_for(origin), slot, xg.at[hop])
            pr = pltpu.make_async_remote_copy(
                src_ref=acc.at[slot],
                dst_ref=precv.at[r - 1],
                send_sem=p_snd.at[r - 1],
                recv_sem=p_rcv.at[r - 1],
                device_id=(origin,),
                device_id_type=pl.DeviceIdType.MESH,
            )
            pr.start()
            part_rdmas[r] = pr

        for q in range(3):
            pltpu.make_async_remote_copy(
                src_ref=acc.at[0], dst_ref=precv.at[q],
                send_sem=p_snd.at[q], recv_sem=p_rcv.at[q],
                device_id=(left,), device_id_type=pl.DeviceIdType.MESH,
            ).wait_recv()
            out_ref[...] = out_ref[...] + precv[q].astype(jnp.float32)

        for r in meta_rdmas:
            r.wait_send()
        for r in hop_rdmas:
            r.wait_send()
        part_rdmas[2].wait_send()
        part_rdmas[3].wait_send()

    return pl.pallas_call(
        body,
        out_shape=jax.ShapeDtypeStruct((N_TOK, D), jnp.float32),
        in_specs=[
            pl.BlockSpec(memory_space=pltpu.VMEM),
            pl.BlockSpec(memory_space=pltpu.VMEM),
            pl.BlockSpec(memory_space=pl.ANY),
        ],
        out_specs=pl.BlockSpec(memory_space=pltpu.VMEM),
        scratch_shapes=[
            pltpu.VMEM((3, N_TOK, D), jnp.bfloat16),
            pltpu.VMEM((3, 2 * E_LOC, N_TOK), jnp.float32),
            pltpu.VMEM((3, N_TOK, D), jnp.bfloat16),
            pltpu.VMEM((2, N_TOK, D), jnp.bfloat16),
            pltpu.VMEM((2, D, D), jnp.bfloat16),
            pltpu.VMEM((2, E_LOC, N_TOK), jnp.float32),
            pltpu.SemaphoreType.DMA((3,)),
            pltpu.SemaphoreType.DMA((3,)),
            pltpu.SemaphoreType.DMA((3,)),
            pltpu.SemaphoreType.DMA((3,)),
            pltpu.SemaphoreType.DMA((3,)),
            pltpu.SemaphoreType.DMA((3,)),
            pltpu.SemaphoreType.DMA((2,)),
        ],
        compiler_params=pltpu.CompilerParams(
            collective_id=0, vmem_limit_bytes=64 * 1024 * 1024),
    )(xb, meta, wb)


# device time: 255887 ns/iter; 1.1907x vs baseline; 1.1183x over previous
import jax
import jax.numpy as jnp
from jax import lax
from jax.experimental import pallas as pl
from jax.experimental.pallas import tpu as pltpu

N_DEV = 4
N_TOK = 2048
D = 1024
E_LOC = 8
E_TOT = 32
CAP = 204.0
K = 768
CAP_PAD = 256


def kernel(x, router_W, route_idx, expert_W):
    del router_W

    me = lax.axis_index("i")
    route = route_idx[:, 0]
    oh = (route[:, None] == jnp.arange(E_TOT, dtype=jnp.int32)[None, :])
    oh = oh.astype(jnp.float32)
    cum = jnp.cumsum(oh, axis=0) - oh
    pos_t = jnp.sum(cum * oh, axis=1)
    le_t = (route % E_LOC).astype(jnp.float32)
    dchip = route // E_LOC
    valid = pos_t < CAP
    xb = x.astype(jnp.bfloat16)

    xd_list, md_list = [], []
    p_all = jnp.zeros((N_TOK,), jnp.int32)
    for d in range(N_DEV):
        sel = (dchip == d) & valid
        p = jnp.cumsum(sel.astype(jnp.int32)) - 1
        idx = jnp.where(sel, p, K)
        xd_list.append(
            jnp.zeros((K, D), jnp.bfloat16).at[idx].set(xb, mode="drop"))
        md_list.append(
            jnp.full((K, 2), -1.0, jnp.float32).at[idx].set(
                jnp.stack([le_t, pos_t], axis=1), mode="drop"))
        p_all = jnp.where(sel, p, p_all)
    xdisp = jnp.stack(xd_list)
    mdisp = jnp.stack(md_list)
    wb = expert_W.astype(jnp.bfloat16)

    def body(xd_ref, md_ref, w_hbm, out_ref,
             rx, rm, rr, acc, wvm,
             x_snd, x_rcv, m_snd, m_rcv, p_snd, p_rcv, w_sem):
        me = lax.axis_index("i")

        bsem = pltpu.get_barrier_semaphore()
        for j in (1, 2, 3):
            pl.semaphore_signal(
                bsem, inc=1, device_id=(lax.rem(me + j, N_DEV),),
                device_id_type=pl.DeviceIdType.MESH)
        pl.semaphore_wait(bsem, 3)

        meta_rd, x_rd = [], []
        for j in (1, 2, 3):
            d = lax.rem(me + j, N_DEV)
            r = pltpu.make_async_remote_copy(
                src_ref=md_ref.at[d], dst_ref=rm.at[3 - j],
                send_sem=m_snd.at[j - 1], recv_sem=m_rcv.at[3 - j],
                device_id=(d,), device_id_type=pl.DeviceIdType.MESH)
            r.start()
            meta_rd.append(r)
        for j in (1, 2, 3):
            d = lax.rem(me + j, N_DEV)
            r = pltpu.make_async_remote_copy(
                src_ref=xd_ref.at[d], dst_ref=rx.at[3 - j],
                send_sem=x_snd.at[j - 1], recv_sem=x_rcv.at[3 - j],
                device_id=(d,), device_id_type=pl.DeviceIdType.MESH)
            r.start()
            x_rd.append(r)

        for r in meta_rd:
            r.wait_recv()

        e_iota = jnp.arange(E_LOC, dtype=jnp.int32)[None, :]
        blks = [md_ref[me]] + [rm[q - 1] for q in (1, 2, 3)]
        shard_ids = [lax.rem(me + q, N_DEV) for q in range(N_DEV)]
        cnts = [jnp.sum((b[:, 0:1].astype(jnp.int32) == e_iota)
                        .astype(jnp.float32), axis=0)
                for b in blks]

        def base_for(q):
            base = jnp.zeros((E_LOC,), jnp.float32)
            for sid, cnt in zip(shard_ids, cnts):
                base = base + jnp.where(sid < shard_ids[q], 1.0, 0.0) * cnt
            return base

        slot_iota = lax.broadcasted_iota(jnp.int32, (CAP_PAD, K), 0)
        e8 = jnp.arange(E_LOC, dtype=jnp.int32)

        def compute_src(q, aslot, xsrc_ref):
            blk = blks[q]
            le_vec = blk[:, 0]
            posv = blk[:, 1]
            base = base_for(q)
            acc[aslot] = jnp.zeros((K, D), jnp.bfloat16)
            pltpu.make_async_copy(w_hbm.at[0], wvm.at[0], w_sem.at[0]).start()

            def le_body(le, _):
                cur = lax.rem(le, 2)
                pltpu.make_async_copy(
                    w_hbm.at[le], wvm.at[cur], w_sem.at[cur]).wait()

                @pl.when(le < E_LOC - 1)
                def _():
                    nxt = lax.rem(le + 1, 2)
                    pltpu.make_async_copy(
                        w_hbm.at[le + 1], wvm.at[nxt], w_sem.at[nxt]).start()

                b_le = jnp.sum(jnp.where(e8 == le, base, 0.0))
                gpos = posv + b_le
                keep = (le_vec.astype(jnp.int32) == le) & (gpos < CAP)
                g = jnp.where(
                    (slot_iota == gpos.astype(jnp.int32)[None, :])
                    & keep[None, :],
                    1.0, 0.0).astype(jnp.bfloat16)
                xc = jnp.dot(g, xsrc_ref[...],
                             preferred_element_type=jnp.float32)
                y = jnp.dot(xc.astype(jnp.bfloat16), wvm[cur],
                            preferred_element_type=jnp.float32)
                sc = lax.dot_general(
                    g, y.astype(jnp.bfloat16), (((0,), (0,)), ((), ())),
                    preferred_element_type=jnp.float32)
                acc[aslot] = acc[aslot] + sc.astype(jnp.bfloat16)
                return 0

            lax.fori_loop(0, E_LOC, le_body, 0)

        compute_src(0, 0, xd_ref.at[me])
        out_ref[0] = acc[0]

        ret_rd = {}
        for q in (1, 2, 3):
            pltpu.make_async_remote_copy(
                src_ref=xd_ref.at[me], dst_ref=rx.at[q - 1],
                send_sem=x_snd.at[q - 1], recv_sem=x_rcv.at[q - 1],
                device_id=(me,), device_id_type=pl.DeviceIdType.MESH,
            ).wait_recv()
            aslot = q % 2
            if q == 3:
                ret_rd[1].wait_send()
            compute_src(q, aslot, rx.at[q - 1])
            o = lax.rem(me + q, N_DEV)
            r = pltpu.make_async_remote_copy(
                src_ref=acc.at[aslot], dst_ref=rr.at[N_DEV - q - 1],
                send_sem=p_snd.at[q - 1], recv_sem=p_rcv.at[3 - q],
                device_id=(o,), device_id_type=pl.DeviceIdType.MESH)
            r.start()
            ret_rd[q] = r

        for jj in range(3):
            pltpu.make_async_remote_copy(
                src_ref=acc.at[0], dst_ref=rr.at[jj],
                send_sem=p_snd.at[jj], recv_sem=p_rcv.at[jj],
                device_id=(me,), device_id_type=pl.DeviceIdType.MESH,
            ).wait_recv()
            out_ref[jj + 1] = rr[jj]

        for r in meta_rd:
            r.wait_send()
        for r in x_rd:
            r.wait_send()
        ret_rd[2].wait_send()
        ret_rd[3].wait_send()

    ret = pl.pallas_call(
        body,
        out_shape=jax.ShapeDtypeStruct((N_DEV, K, D), jnp.bfloat16),
        in_specs=[
            pl.BlockSpec(memory_space=pltpu.VMEM),
            pl.BlockSpec(memory_space=pltpu.VMEM),
            pl.BlockSpec(memory_space=pl.ANY),
        ],
        out_specs=pl.BlockSpec(memory_space=pltpu.VMEM),
        scratch_shapes=[
            pltpu.VMEM((3, K, D), jnp.bfloat16),
            pltpu.VMEM((3, K, 2), jnp.float32),
            pltpu.VMEM((3, K, D), jnp.bfloat16),
            pltpu.VMEM((2, K, D), jnp.bfloat16),
            pltpu.VMEM((2, D, D), jnp.bfloat16),
            pltpu.SemaphoreType.DMA((3,)),
            pltpu.SemaphoreType.DMA((3,)),
            pltpu.SemaphoreType.DMA((3,)),
            pltpu.SemaphoreType.DMA((3,)),
            pltpu.SemaphoreType.DMA((3,)),
            pltpu.SemaphoreType.DMA((3,)),
            pltpu.SemaphoreType.DMA((2,)),
        ],
        compiler_params=pltpu.CompilerParams(
            collective_id=0, vmem_limit_bytes=64 * 1024 * 1024),
    )(xdisp, mdisp, wb)

    rel = jnp.remainder(dchip - me, N_DEV)
    fidx = jnp.where(valid, rel * K + p_all, 0)
    rows = jnp.take(ret.reshape(N_DEV * K, D), fidx, axis=0)
    return jnp.where(valid[:, None], rows, 0).astype(jnp.float32)


# device time: 210439 ns/iter; 1.4478x vs baseline; 1.2160x over previous
import jax
import jax.numpy as jnp
from jax import lax
from jax.experimental import pallas as pl
from jax.experimental.pallas import tpu as pltpu

N_DEV = 4
N_TOK = 2048
D = 1024
E_LOC = 8
E_TOT = 32
CAP = 204.0
K = 768
CAP_PAD = 256


def kernel(x, router_W, route_idx, expert_W):
    del router_W

    me = lax.axis_index("i")
    route = route_idx[:, 0]
    oh = (route[:, None] == jnp.arange(E_TOT, dtype=jnp.int32)[None, :])
    oh = oh.astype(jnp.float32)
    cum = jnp.cumsum(oh, axis=0) - oh
    pos_t = jnp.sum(cum * oh, axis=1)
    le_t = (route % E_LOC).astype(jnp.float32)
    dchip = route // E_LOC
    valid = pos_t < CAP
    xb = x.astype(jnp.bfloat16)

    le_enc = le_t + 1.0
    lepos = jnp.stack([le_enc, pos_t], axis=1)
    pv_list = []
    for d in range(N_DEV):
        sel = (dchip == d) & valid
        p = jnp.cumsum(sel.astype(jnp.int32)) - 1
        pv_list.append(jnp.where(sel, p, -1).astype(jnp.float32))
    pvec = jnp.stack(pv_list)
    wb = expert_W.astype(jnp.bfloat16)

    def body(xb_ref, pvec_ref, lepos_ref, w_hbm, out_ref,
             xd_ref, md_ref, rx, rm, rr, acc, wvm,
             x_snd, x_rcv, m_snd, m_rcv, p_snd, p_rcv, w_sem):
        me = lax.axis_index("i")

        row_iota = lax.broadcasted_iota(jnp.int32, (K, N_TOK), 0)
        for d in range(N_DEV):
            pvi = pvec_ref[d].astype(jnp.int32)
            gd = (row_iota == pvi[None, :]).astype(jnp.bfloat16)
            xd_ref[d] = jnp.dot(
                gd, xb_ref[...],
                preferred_element_type=jnp.float32).astype(jnp.bfloat16)
            md_ref[d] = jnp.dot(gd, lepos_ref[...],
                                preferred_element_type=jnp.float32)

        bsem = pltpu.get_barrier_semaphore()
        for j in (1, 2, 3):
            pl.semaphore_signal(
                bsem, inc=1, device_id=(lax.rem(me + j, N_DEV),),
                device_id_type=pl.DeviceIdType.MESH)
        pl.semaphore_wait(bsem, 3)

        meta_rd, x_rd = [], []
        for j in (1, 2, 3):
            d = lax.rem(me + j, N_DEV)
            r = pltpu.make_async_remote_copy(
                src_ref=md_ref.at[d], dst_ref=rm.at[3 - j],
                send_sem=m_snd.at[j - 1], recv_sem=m_rcv.at[3 - j],
                device_id=(d,), device_id_type=pl.DeviceIdType.MESH)
            r.start()
            meta_rd.append(r)
        for j in (1, 2, 3):
            d = lax.rem(me + j, N_DEV)
            r = pltpu.make_async_remote_copy(
                src_ref=xd_ref.at[d], dst_ref=rx.at[3 - j],
                send_sem=x_snd.at[j - 1], recv_sem=x_rcv.at[3 - j],
                device_id=(d,), device_id_type=pl.DeviceIdType.MESH)
            r.start()
            x_rd.append(r)

        for r in meta_rd:
            r.wait_recv()

        e_iota1 = jnp.arange(E_LOC, dtype=jnp.int32)[None, :] + 1
        blks = [md_ref[me]] + [rm[q - 1] for q in (1, 2, 3)]
        shard_ids = [lax.rem(me + q, N_DEV) for q in range(N_DEV)]
        cnts = [jnp.sum((b[:, 0:1].astype(jnp.int32) == e_iota1)
                        .astype(jnp.float32), axis=0)
                for b in blks]

        def base_for(q):
            base = jnp.zeros((E_LOC,), jnp.float32)
            for sid, cnt in zip(shard_ids, cnts):
                base = base + jnp.where(sid < shard_ids[q], 1.0, 0.0) * cnt
            return base

        slot_iota = lax.broadcasted_iota(jnp.int32, (CAP_PAD, K), 0)
        e8 = jnp.arange(E_LOC, dtype=jnp.int32)

        def compute_src(q, aslot, xsrc_ref):
            blk = blks[q]
            le_vec = blk[:, 0]
            posv = blk[:, 1]
            base = base_for(q)
            acc[aslot] = jnp.zeros((K, D), jnp.bfloat16)
            pltpu.make_async_copy(w_hbm.at[0], wvm.at[0], w_sem.at[0]).start()

            def le_body(le, _):
                cur = lax.rem(le, 2)
                pltpu.make_async_copy(
                    w_hbm.at[le], wvm.at[cur], w_sem.at[cur]).wait()

                @pl.when(le < E_LOC - 1)
                def _():
                    nxt = lax.rem(le + 1, 2)
                    pltpu.make_async_copy(
                        w_hbm.at[le + 1], wvm.at[nxt], w_sem.at[nxt]).start()

                b_le = jnp.sum(jnp.where(e8 == le, base, 0.0))
                gpos = posv + b_le
                keep = (le_vec.astype(jnp.int32) == le + 1) & (gpos < CAP)
                g = jnp.where(
                    (slot_iota == gpos.astype(jnp.int32)[None, :])
                    & keep[None, :],
                    1.0, 0.0).astype(jnp.bfloat16)
                xc = jnp.dot(g, xsrc_ref[...],
                             preferred_element_type=jnp.float32)
                y = jnp.dot(xc.astype(jnp.bfloat16), wvm[cur],
                            preferred_element_type=jnp.float32)
                sc = lax.dot_general(
                    g, y.astype(jnp.bfloat16), (((0,), (0,)), ((), ())),
                    preferred_element_type=jnp.float32)
                acc[aslot] = acc[aslot] + sc.astype(jnp.bfloat16)
                return 0

            lax.fori_loop(0, E_LOC, le_body, 0)

        compute_src(0, 0, xd_ref.at[me])

        col_iota = lax.broadcasted_iota(jnp.int32, (N_TOK, K), 1)

        def assemble(pv_idx, block):
            pvi = pvec_ref[pv_idx].astype(jnp.int32)
            s = (col_iota == pvi[:, None]).astype(jnp.bfloat16)
            return jnp.dot(s, block, preferred_element_type=jnp.float32)

        out_ref[...] = assemble(me, acc[0])

        ret_rd = {}
        for q in (1, 2, 3):
            pltpu.make_async_remote_copy(
                src_ref=xd_ref.at[me], dst_ref=rx.at[q - 1],
                send_sem=x_snd.at[q - 1], recv_sem=x_rcv.at[q - 1],
                device_id=(me,), device_id_type=pl.DeviceIdType.MESH,
            ).wait_recv()
            aslot = q % 2
            if q == 3:
                ret_rd[1].wait_send()
            compute_src(q, aslot, rx.at[q - 1])
            o = lax.rem(me + q, N_DEV)
            r = pltpu.make_async_remote_copy(
                src_ref=acc.at[aslot], dst_ref=rr.at[N_DEV - q - 1],
                send_sem=p_snd.at[q - 1], recv_sem=p_rcv.at[3 - q],
                device_id=(o,), device_id_type=pl.DeviceIdType.MESH)
            r.start()
            ret_rd[q] = r

        for jj in range(3):
            pltpu.make_async_remote_copy(
                src_ref=acc.at[0], dst_ref=rr.at[jj],
                send_sem=p_snd.at[jj], recv_sem=p_rcv.at[jj],
                device_id=(me,), device_id_type=pl.DeviceIdType.MESH,
            ).wait_recv()
            out_ref[...] = out_ref[...] + assemble(
                lax.rem(me + jj + 1, N_DEV), rr[jj])

        for r in meta_rd:
            r.wait_send()
        for r in x_rd:
            r.wait_send()
        ret_rd[2].wait_send()
        ret_rd[3].wait_send()

    return pl.pallas_call(
        body,
        out_shape=jax.ShapeDtypeStruct((N_TOK, D), jnp.float32),
        in_specs=[
            pl.BlockSpec(memory_space=pltpu.VMEM),
            pl.BlockSpec(memory_space=pltpu.VMEM),
            pl.BlockSpec(memory_space=pltpu.VMEM),
            pl.BlockSpec(memory_space=pl.ANY),
        ],
        out_specs=pl.BlockSpec(memory_space=pltpu.VMEM),
        scratch_shapes=[
            pltpu.VMEM((N_DEV, K, D), jnp.bfloat16),
            pltpu.VMEM((N_DEV, K, 2), jnp.float32),
            pltpu.VMEM((3, K, D), jnp.bfloat16),
            pltpu.VMEM((3, K, 2), jnp.float32),
            pltpu.VMEM((3, K, D), jnp.bfloat16),
            pltpu.VMEM((2, K, D), jnp.bfloat16),
            pltpu.VMEM((2, D, D), jnp.bfloat16),
            pltpu.SemaphoreType.DMA((3,)),
            pltpu.SemaphoreType.DMA((3,)),
            pltpu.SemaphoreType.DMA((3,)),
            pltpu.SemaphoreType.DMA((3,)),
            pltpu.SemaphoreType.DMA((3,)),
            pltpu.SemaphoreType.DMA((3,)),
            pltpu.SemaphoreType.DMA((2,)),
        ],
        compiler_params=pltpu.CompilerParams(
            collective_id=0, vmem_limit_bytes=64 * 1024 * 1024),
    )(xb, pvec, lepos, wb)
